# baseline (speedup 1.0000x reference)
import jax
import jax.numpy as jnp
from jax import lax
from jax.experimental import pallas as pl
from jax.experimental.pallas import tpu as pltpu

H = 2048
E = 16
NF = 16384
FE = NF // E

CE = 4
CD = 4
HB = H // CE
WB = FE // CD


def _body(act_ref, eb_ref, acol_ref, rbcol_ref, router_ref, enc_hbm, dec_hbm,
          out_ref, enc_buf, dec_buf, enc_sems, dec_sems):
    xrc = acol_ref[...] - rbcol_ref[...]
    logits = jnp.sum(xrc * router_ref[...],
                     axis=0, keepdims=True)
    m = jnp.max(logits)
    maxp = 1.0 / jnp.sum(jnp.exp(logits - m))
    iota = lax.broadcasted_iota(jnp.int32, (1, E), 1)
    idx = jnp.min(jnp.where(logits == m, iota, E))

    enc_copies = [
        pltpu.make_async_copy(
            enc_hbm.at[idx, pl.ds(k * HB, HB), :],
            enc_buf.at[pl.ds(k * HB, HB), :],
            enc_sems.at[k],
        )
        for k in range(CE)
    ]
    dec_copies = [
        pltpu.make_async_copy(
            dec_hbm.at[idx, pl.ds(k * WB, WB), :],
            dec_buf.at[pl.ds(k * WB, WB), :],
            dec_sems.at[k],
        )
        for k in range(CD)
    ]
    for c in enc_copies:
        c.start()
    for c in dec_copies:
        c.start()

    x = act_ref[...] - eb_ref[...]
    f = None
    for k in range(CE):
        enc_copies[k].wait()
        pf = jnp.dot(x[:, k * HB:(k + 1) * HB],
                     enc_buf[k * HB:(k + 1) * HB, :],
                     preferred_element_type=jnp.float32)
        f = pf if f is None else f + pf
    f = jnp.maximum(f, 0.0)

    acc = None
    for k in range(CD):
        dec_copies[k].wait()
        c = jnp.dot(f[:, k * WB:(k + 1) * WB],
                    dec_buf[k * WB:(k + 1) * WB, :],
                    preferred_element_type=jnp.float32)
        acc = c if acc is None else acc + c

    out_ref[...] = acc * maxp + eb_ref[...]


def kernel(activations, enc, dec, expert_b, router_b, router):
    act2 = activations.reshape(1, H)
    eb2 = expert_b.reshape(1, H)
    acol = activations.reshape(H, 1)
    rbcol = router_b.reshape(H, 1)

    out = pl.pallas_call(
        _body,
        in_specs=[
            pl.BlockSpec(memory_space=pltpu.VMEM),
            pl.BlockSpec(memory_space=pltpu.VMEM),
            pl.BlockSpec(memory_space=pltpu.VMEM),
            pl.BlockSpec(memory_space=pltpu.VMEM),
            pl.BlockSpec(memory_space=pltpu.VMEM),
            pl.BlockSpec(memory_space=pl.ANY),
            pl.BlockSpec(memory_space=pl.ANY),
        ],
        out_specs=pl.BlockSpec(memory_space=pltpu.VMEM),
        out_shape=jax.ShapeDtypeStruct((1, H), jnp.float32),
        scratch_shapes=[
            pltpu.VMEM((H, FE), jnp.float32),
            pltpu.VMEM((FE, H), jnp.float32),
            pltpu.SemaphoreType.DMA((CE,)),
            pltpu.SemaphoreType.DMA((CD,)),
        ],
    )(act2, eb2, acol, rbcol, router, enc, dec)

    return out.reshape(H)


# speedup vs baseline: 1.4375x; 1.4375x over previous
import jax
import jax.numpy as jnp
from jax import lax
from jax.experimental import pallas as pl
from jax.experimental.pallas import tpu as pltpu

H = 2048
E = 16
NF = 16384
FE = NF // E

CE = 4
CD = 4
HB = H // CE
WB = FE // CD


def _body(act_ref, eb_ref, rb_ref, router_ref, enc_hbm, dec_hbm, out_ref,
          enc_buf, dec_buf, enc_sems, dec_sems):
    xr = act_ref[...] - rb_ref[...]
    logits = jnp.dot(xr, router_ref[...],
                     preferred_element_type=jnp.float32)
    m = jnp.max(logits)
    maxp = 1.0 / jnp.sum(jnp.exp(logits - m))
    iota = lax.broadcasted_iota(jnp.int32, (1, E), 1)
    idx = jnp.min(jnp.where(logits == m, iota, E))

    enc_copies = [
        pltpu.make_async_copy(
            enc_hbm.at[idx, pl.ds(k * HB, HB), :],
            enc_buf.at[pl.ds(k * HB, HB), :],
            enc_sems.at[k],
        )
        for k in range(CE)
    ]
    dec_copies = [
        pltpu.make_async_copy(
            dec_hbm.at[idx, pl.ds(k * WB, WB), :],
            dec_buf.at[pl.ds(k * WB, WB), :],
            dec_sems.at[k],
        )
        for k in range(CD)
    ]
    for c in enc_copies:
        c.start()
    for c in dec_copies:
        c.start()

    x = act_ref[...] - eb_ref[...]
    f = None
    for k in range(CE):
        enc_copies[k].wait()
        pf = jnp.dot(x[:, k * HB:(k + 1) * HB],
                     enc_buf[k * HB:(k + 1) * HB, :],
                     preferred_element_type=jnp.float32)
        f = pf if f is None else f + pf
    f = jnp.maximum(f, 0.0)

    acc = None
    for k in range(CD):
        dec_copies[k].wait()
        c = jnp.dot(f[:, k * WB:(k + 1) * WB],
                    dec_buf[k * WB:(k + 1) * WB, :],
                    preferred_element_type=jnp.float32)
        acc = c if acc is None else acc + c

    out_ref[...] = acc * maxp + eb_ref[...]


def kernel(activations, enc, dec, expert_b, router_b, router):
    act2 = activations.reshape(1, H)
    rb2 = router_b.reshape(1, H)
    eb2 = expert_b.reshape(1, H)

    out = pl.pallas_call(
        _body,
        in_specs=[
            pl.BlockSpec(memory_space=pltpu.VMEM),
            pl.BlockSpec(memory_space=pltpu.VMEM),
            pl.BlockSpec(memory_space=pltpu.VMEM),
            pl.BlockSpec(memory_space=pltpu.VMEM),
            pl.BlockSpec(memory_space=pl.ANY),
            pl.BlockSpec(memory_space=pl.ANY),
        ],
        out_specs=pl.BlockSpec(memory_space=pltpu.VMEM),
        out_shape=jax.ShapeDtypeStruct((1, H), jnp.float32),
        scratch_shapes=[
            pltpu.VMEM((H, FE), jnp.float32),
            pltpu.VMEM((FE, H), jnp.float32),
            pltpu.SemaphoreType.DMA((CE,)),
            pltpu.SemaphoreType.DMA((CD,)),
        ],
    )(act2, eb2, rb2, router, enc, dec)

    return out.reshape(H)
